# baseline (speedup 1.0000x reference)
import functools

import jax
import jax.numpy as jnp
from jax import lax
from jax.experimental import pallas as pl
from jax.experimental.pallas import tpu as pltpu
from jax.experimental.pallas import tpu_sc as plsc

_NUM_CORES = 2
_NUM_SUBCORES = 16
_NUM_WORKERS = _NUM_CORES * _NUM_SUBCORES

_CHUNK = 32
_NBUF = 3


@functools.partial(jax.jit, static_argnames=("n_rows", "hidden"))
def _lookup(positions2d, table, *, n_rows, hidden):
    per_w = n_rows // _NUM_WORKERS
    n_chunks = per_w // _CHUNK
    mesh = plsc.VectorSubcoreMesh(core_axis_name="c", subcore_axis_name="s")

    @functools.partial(
        pl.kernel,
        mesh=mesh,
        out_type=jax.ShapeDtypeStruct((n_rows, hidden), jnp.float32),
        scratch_types=(
            [pltpu.VMEM((n_chunks, _CHUNK), jnp.int32)]
            + [pltpu.VMEM((_CHUNK, hidden), jnp.float32)] * _NBUF
            + [pltpu.SemaphoreType.DMA] * (2 * _NBUF)
        ),
    )
    def emb_kernel(idx_hbm, table_hbm, out_hbm, idx_v, *bufs):
        rows = bufs[:_NBUF]
        gsem = bufs[_NBUF:2 * _NBUF]
        osem = bufs[2 * _NBUF:]
        wid = lax.axis_index("s") * _NUM_CORES + lax.axis_index("c")
        base = wid * per_w
        chunk_row = wid * n_chunks

        pltpu.sync_copy(idx_hbm.at[pl.ds(chunk_row, n_chunks)], idx_v)

        gcp = [None] * n_chunks
        ocp = [None] * n_chunks

        def writeback(g):
            b = g % _NBUF
            gcp[g].wait()
            ocp[g] = pltpu.async_copy(
                rows[b], out_hbm.at[pl.ds(base + g * _CHUNK, _CHUNK)], osem[b])

        for g in range(n_chunks):
            b = g % _NBUF
            if g >= _NBUF:
                ocp[g - _NBUF].wait()
            gcp[g] = pltpu.async_copy(table_hbm.at[idx_v.at[g]], rows[b], gsem[b])
            if g >= 1:
                writeback(g - 1)

        writeback(n_chunks - 1)
        for g in range(max(0, n_chunks - _NBUF), n_chunks):
            ocp[g].wait()

    return emb_kernel(positions2d, table)


def kernel(positions, pos_embedding):
    b, t = positions.shape
    n_rows = b * t
    hidden = pos_embedding.shape[1]
    pos2d = positions.reshape(n_rows // _CHUNK, _CHUNK).astype(jnp.int32)
    out = _lookup(pos2d, pos_embedding, n_rows=n_rows, hidden=hidden)
    return out.reshape(b, t, hidden)
